# Initial kernel scaffold; baseline (speedup 1.0000x reference)
import jax
import jax.numpy as jnp
from jax.experimental import pallas as pl


def kernel(h_subgraph, subgraph_idx_batch, W_fc0, b_fc0, W_fc1, b_fc1, W_sum0, b_sum0, W_sum1, b_sum1, W_out1, b_out1, W_out2, b_out2):
    raise NotImplementedError("write your pallas kernel here")


# speedup vs baseline: 2.4799x; 2.4799x over previous
import jax
import jax.numpy as jnp
from jax import lax
from jax.experimental import pallas as pl
from jax.experimental.pallas import tpu as pltpu
from jax.experimental.pallas import tpu_sc as plsc

_N = 320000
_D = 128
_G = 5000
_T = 10

_G_PAD = 5120
_PAD_SEG = _G
_N_PAD = 327680
_BLK_A = 640
_CH = 128
_SUB = 128
_BLK_B = 512


def _elu(x):
    return jnp.where(x > 0.0, x, jnp.exp(jnp.minimum(x, 0.0)) - 1.0)


def _dense_body(x_ref, w0_ref, b0_ref, w1_ref, b1_ref, out_ref):
    x = x_ref[...].astype(jnp.bfloat16)
    w0 = w0_ref[...].astype(jnp.bfloat16)
    a = jnp.dot(x, w0, preferred_element_type=jnp.float32)
    a = _elu(a + b0_ref[...])
    w1 = w1_ref[...].astype(jnp.bfloat16)
    a = jnp.dot(a.astype(jnp.bfloat16), w1, preferred_element_type=jnp.float32)
    out_ref[...] = _elu(a + b1_ref[...])


def _dense(h, w0, b0, w1, b1):
    nblk_in = _N // _BLK_A
    grid = _N_PAD // _BLK_A
    return pl.pallas_call(
        _dense_body,
        grid=(grid,),
        in_specs=[
            pl.BlockSpec((_BLK_A, _D), lambda i: (jnp.minimum(i, nblk_in - 1), 0)),
            pl.BlockSpec((_D, _D), lambda i: (0, 0)),
            pl.BlockSpec((1, _D), lambda i: (0, 0)),
            pl.BlockSpec((_D, _D), lambda i: (0, 0)),
            pl.BlockSpec((1, _D), lambda i: (0, 0)),
        ],
        out_specs=pl.BlockSpec((_BLK_A, _D), lambda i: (i, 0)),
        out_shape=jax.ShapeDtypeStruct((_N_PAD, _D), jnp.float32),
    )(h, w0, b0.reshape(1, _D), w1, b1.reshape(1, _D))


def _seg_body(act, idx2d, zsum, ones, out_sum, out_cnt,
              shsum, shcnt, buf_a, buf_b, idx_v, ones_v,
              sem_a, sem_b):
    c = lax.axis_index("c")
    s = lax.axis_index("s")
    rows_per_tile = _N_PAD // 32
    base = (c * 16 + s) * rows_per_tile
    gsl = pl.ds(s * (_G_PAD // 16), _G_PAD // 16)

    pltpu.sync_copy(zsum, shsum.at[gsl])
    pltpu.sync_copy(zsum, shcnt.at[gsl])
    pltpu.sync_copy(ones, ones_v)
    plsc.subcore_barrier()

    def scatter(buf, q):
        pltpu.sync_copy(buf, shsum.at[idx_v.at[q]], add=True)
        pltpu.sync_copy(ones_v, shcnt.at[idx_v.at[q, pl.ds(0, 64)]], add=True)
        pltpu.sync_copy(ones_v, shcnt.at[idx_v.at[q, pl.ds(64, 64)]], add=True)

    npairs = rows_per_tile // (2 * _CH)

    pltpu.async_copy(act.at[pl.ds(pl.multiple_of(base, 8), _CH)], buf_a, sem_a)

    def pair_body(p, carry):
        off = pl.multiple_of(base + p * (2 * _CH), 8)

        @pl.when(lax.rem(p, 4) == 0)
        def _():
            irow = pl.multiple_of(base // 128 + (p // 4) * 8, 8)
            pltpu.sync_copy(idx2d.at[pl.ds(irow, 8)], idx_v)

        q = lax.rem(p, 4) * 2
        pltpu.make_async_copy(act.at[pl.ds(off, _CH)], buf_a, sem_a).wait()
        pltpu.async_copy(act.at[pl.ds(off + _CH, _CH)], buf_b, sem_b)
        scatter(buf_a, q)
        pltpu.make_async_copy(act.at[pl.ds(off, _CH)], buf_b, sem_b).wait()

        @pl.when(p < npairs - 1)
        def _():
            nxt = pl.multiple_of(off + 2 * _CH, 8)
            pltpu.async_copy(act.at[pl.ds(nxt, _CH)], buf_a, sem_a)

        scatter(buf_b, q + 1)
        return carry

    lax.fori_loop(0, npairs, pair_body, 0)
    plsc.subcore_barrier()

    pltpu.sync_copy(shsum.at[gsl], out_sum.at[c, gsl])
    pltpu.sync_copy(shcnt.at[gsl], out_cnt.at[c, gsl])


def _segment_sums(act, idx2d):
    zsum = jnp.zeros((_G_PAD // 16, _D), jnp.float32)
    ones = jnp.ones((64, _D), jnp.float32)
    mesh = plsc.VectorSubcoreMesh(core_axis_name="c", subcore_axis_name="s")
    return pl.kernel(
        _seg_body,
        out_type=[
            jax.ShapeDtypeStruct((2, _G_PAD, _D), jnp.float32),
            jax.ShapeDtypeStruct((2, _G_PAD, _D), jnp.float32),
        ],
        mesh=mesh,
        scratch_types=[
            pltpu.VMEM_SHARED((_G_PAD, _D), jnp.float32),
            pltpu.VMEM_SHARED((_G_PAD, _D), jnp.float32),
            pltpu.VMEM((_CH, _D), jnp.float32),
            pltpu.VMEM((_CH, _D), jnp.float32),
            pltpu.VMEM((8, 128), jnp.int32),
            pltpu.VMEM((64, _D), jnp.float32),
            pltpu.SemaphoreType.DMA,
            pltpu.SemaphoreType.DMA,
        ],
    )(act, idx2d, zsum, ones)


def _head_body(sum_ref, cnt_ref, ws0_ref, bs0_ref, ws1_ref, bs1_ref,
               wo1_ref, bo1_ref, wo2_ref, bo2_ref, out_ref):
    ssum = sum_ref[0] + sum_ref[1]
    cnt = cnt_ref[0] + cnt_ref[1]
    mean = ssum / jnp.maximum(cnt, 1.0)
    g = jnp.dot(mean, ws0_ref[...], preferred_element_type=jnp.float32)
    g = _elu(g + bs0_ref[...])
    g = jnp.dot(g, ws1_ref[...], preferred_element_type=jnp.float32)
    g = _elu(g + bs1_ref[...])
    r = jnp.dot(g, wo1_ref[...], preferred_element_type=jnp.float32)
    r = jnp.maximum(r + bo1_ref[...], 0.0)
    o = jnp.dot(r, wo2_ref[...], preferred_element_type=jnp.float32)
    out_ref[...] = o + bo2_ref[...]


def _head(sums, cnts, ws0, bs0, ws1, bs1, wo1, bo1, wo2p, bo2p):
    grid = _G_PAD // _BLK_B
    return pl.pallas_call(
        _head_body,
        grid=(grid,),
        in_specs=[
            pl.BlockSpec((2, _BLK_B, _D), lambda i: (0, i, 0)),
            pl.BlockSpec((2, _BLK_B, _D), lambda i: (0, i, 0)),
            pl.BlockSpec((_D, _D), lambda i: (0, 0)),
            pl.BlockSpec((1, _D), lambda i: (0, 0)),
            pl.BlockSpec((_D, _D), lambda i: (0, 0)),
            pl.BlockSpec((1, _D), lambda i: (0, 0)),
            pl.BlockSpec((_D, 256), lambda i: (0, 0)),
            pl.BlockSpec((1, 256), lambda i: (0, 0)),
            pl.BlockSpec((256, 128), lambda i: (0, 0)),
            pl.BlockSpec((1, 128), lambda i: (0, 0)),
        ],
        out_specs=pl.BlockSpec((_BLK_B, 128), lambda i: (i, 0)),
        out_shape=jax.ShapeDtypeStruct((_G_PAD, 128), jnp.float32),
    )(sums, cnts, ws0, bs0.reshape(1, _D), ws1, bs1.reshape(1, _D),
      wo1, bo1.reshape(1, 256), wo2p, bo2p)


def kernel(h_subgraph, subgraph_idx_batch, W_fc0, b_fc0, W_fc1, b_fc1,
           W_sum0, b_sum0, W_sum1, b_sum1, W_out1, b_out1, W_out2, b_out2):
    act = _dense(h_subgraph, W_fc0, b_fc0, W_fc1, b_fc1)

    idx_pad = jnp.concatenate(
        [subgraph_idx_batch,
         jnp.full((_N_PAD - _N,), _PAD_SEG, jnp.int32)])
    idx2d = idx_pad.reshape(_N_PAD // 128, 128)
    sums, cnts = _segment_sums(act, idx2d)

    wo2p = jnp.zeros((256, 128), jnp.float32).at[:, :_T].set(W_out2)
    bo2p = jnp.zeros((1, 128), jnp.float32).at[0, :_T].set(b_out2)
    out = _head(sums, cnts, W_sum0, b_sum0, W_sum1, b_sum1,
                W_out1, b_out1, wo2p, bo2p)
    return out[:_G, :_T]


# speedup vs baseline: 3.7925x; 1.5293x over previous
import jax
import jax.numpy as jnp
from jax import lax
from jax.experimental import pallas as pl
from jax.experimental.pallas import tpu as pltpu
from jax.experimental.pallas import tpu_sc as plsc

_N = 320000
_D = 128
_G = 5000
_T = 10

_G_PAD = 5120
_PAD_SEG = _G
_N_PAD = 327680
_BLK_A = 2560
_CH = 128
_SUB = 128
_BLK_B = 512


def _elu(x):
    return jnp.where(x > 0.0, x, jnp.exp(jnp.minimum(x, 0.0)) - 1.0)


def _dense_body(x_ref, w0_ref, b0_ref, w1_ref, b1_ref, out_ref):
    x = x_ref[...].astype(jnp.bfloat16)
    w0 = w0_ref[...].astype(jnp.bfloat16)
    a = jnp.dot(x, w0, preferred_element_type=jnp.float32)
    a = _elu(a + b0_ref[...])
    w1 = w1_ref[...].astype(jnp.bfloat16)
    a = jnp.dot(a.astype(jnp.bfloat16), w1, preferred_element_type=jnp.float32)
    out_ref[...] = _elu(a + b1_ref[...])


def _dense(h, w0, b0, w1, b1):
    nblk_in = _N // _BLK_A
    grid = _N_PAD // _BLK_A
    return pl.pallas_call(
        _dense_body,
        grid=(grid,),
        in_specs=[
            pl.BlockSpec((_BLK_A, _D), lambda i: (jnp.minimum(i, nblk_in - 1), 0)),
            pl.BlockSpec((_D, _D), lambda i: (0, 0)),
            pl.BlockSpec((1, _D), lambda i: (0, 0)),
            pl.BlockSpec((_D, _D), lambda i: (0, 0)),
            pl.BlockSpec((1, _D), lambda i: (0, 0)),
        ],
        out_specs=pl.BlockSpec((_BLK_A, _D), lambda i: (i, 0)),
        out_shape=jax.ShapeDtypeStruct((_N_PAD, _D), jnp.float32),
    )(h, w0, b0.reshape(1, _D), w1, b1.reshape(1, _D))


def _seg_body(act, idx2d, zsum, ones, out_sum, out_cnt,
              shsum, shcnt, buf_a, buf_b, idx_v, ones_v,
              sem_a, sem_b):
    c = lax.axis_index("c")
    s = lax.axis_index("s")
    rows_per_tile = _N_PAD // 32
    base = (c * 16 + s) * rows_per_tile
    gsl = pl.ds(s * (_G_PAD // 16), _G_PAD // 16)

    pltpu.sync_copy(zsum, shsum.at[gsl])
    pltpu.sync_copy(zsum, shcnt.at[gsl])
    pltpu.sync_copy(ones, ones_v)
    plsc.subcore_barrier()

    def scatter(buf, q):
        pltpu.sync_copy(buf, shsum.at[idx_v.at[q]], add=True)
        pltpu.sync_copy(ones_v, shcnt.at[idx_v.at[q, pl.ds(0, 64)]], add=True)
        pltpu.sync_copy(ones_v, shcnt.at[idx_v.at[q, pl.ds(64, 64)]], add=True)

    npairs = rows_per_tile // (2 * _CH)

    pltpu.async_copy(act.at[pl.ds(pl.multiple_of(base, 8), _CH)], buf_a, sem_a)

    def pair_body(p, carry):
        off = pl.multiple_of(base + p * (2 * _CH), 8)

        @pl.when(lax.rem(p, 4) == 0)
        def _():
            irow = pl.multiple_of(base // 128 + (p // 4) * 8, 8)
            pltpu.sync_copy(idx2d.at[pl.ds(irow, 8)], idx_v)

        q = lax.rem(p, 4) * 2
        pltpu.make_async_copy(act.at[pl.ds(off, _CH)], buf_a, sem_a).wait()
        pltpu.async_copy(act.at[pl.ds(off + _CH, _CH)], buf_b, sem_b)
        scatter(buf_a, q)
        pltpu.make_async_copy(act.at[pl.ds(off, _CH)], buf_b, sem_b).wait()

        @pl.when(p < npairs - 1)
        def _():
            nxt = pl.multiple_of(off + 2 * _CH, 8)
            pltpu.async_copy(act.at[pl.ds(nxt, _CH)], buf_a, sem_a)

        scatter(buf_b, q + 1)
        return carry

    lax.fori_loop(0, npairs, pair_body, 0)
    plsc.subcore_barrier()

    pltpu.sync_copy(shsum.at[gsl], out_sum.at[c, gsl])
    pltpu.sync_copy(shcnt.at[gsl], out_cnt.at[c, gsl])


def _segment_sums(act, idx2d):
    zsum = jnp.zeros((_G_PAD // 16, _D), jnp.float32)
    ones = jnp.ones((64, _D), jnp.float32)
    mesh = plsc.VectorSubcoreMesh(core_axis_name="c", subcore_axis_name="s")
    return pl.kernel(
        _seg_body,
        out_type=[
            jax.ShapeDtypeStruct((2, _G_PAD, _D), jnp.float32),
            jax.ShapeDtypeStruct((2, _G_PAD, _D), jnp.float32),
        ],
        mesh=mesh,
        scratch_types=[
            pltpu.VMEM_SHARED((_G_PAD, _D), jnp.float32),
            pltpu.VMEM_SHARED((_G_PAD, _D), jnp.float32),
            pltpu.VMEM((_CH, _D), jnp.float32),
            pltpu.VMEM((_CH, _D), jnp.float32),
            pltpu.VMEM((8, 128), jnp.int32),
            pltpu.VMEM((64, _D), jnp.float32),
            pltpu.SemaphoreType.DMA,
            pltpu.SemaphoreType.DMA,
        ],
    )(act, idx2d, zsum, ones)


def _head_body(sum_ref, cnt_ref, ws0_ref, bs0_ref, ws1_ref, bs1_ref,
               wo1_ref, bo1_ref, wo2_ref, bo2_ref, out_ref):
    ssum = sum_ref[0] + sum_ref[1]
    cnt = cnt_ref[0] + cnt_ref[1]
    mean = ssum / jnp.maximum(cnt, 1.0)
    g = jnp.dot(mean, ws0_ref[...], preferred_element_type=jnp.float32)
    g = _elu(g + bs0_ref[...])
    g = jnp.dot(g, ws1_ref[...], preferred_element_type=jnp.float32)
    g = _elu(g + bs1_ref[...])
    r = jnp.dot(g, wo1_ref[...], preferred_element_type=jnp.float32)
    r = jnp.maximum(r + bo1_ref[...], 0.0)
    o = jnp.dot(r, wo2_ref[...], preferred_element_type=jnp.float32)
    out_ref[...] = o + bo2_ref[...]


def _head(sums, cnts, ws0, bs0, ws1, bs1, wo1, bo1, wo2p, bo2p):
    grid = _G_PAD // _BLK_B
    return pl.pallas_call(
        _head_body,
        grid=(grid,),
        in_specs=[
            pl.BlockSpec((2, _BLK_B, _D), lambda i: (0, i, 0)),
            pl.BlockSpec((2, _BLK_B, _D), lambda i: (0, i, 0)),
            pl.BlockSpec((_D, _D), lambda i: (0, 0)),
            pl.BlockSpec((1, _D), lambda i: (0, 0)),
            pl.BlockSpec((_D, _D), lambda i: (0, 0)),
            pl.BlockSpec((1, _D), lambda i: (0, 0)),
            pl.BlockSpec((_D, 256), lambda i: (0, 0)),
            pl.BlockSpec((1, 256), lambda i: (0, 0)),
            pl.BlockSpec((256, 128), lambda i: (0, 0)),
            pl.BlockSpec((1, 128), lambda i: (0, 0)),
        ],
        out_specs=pl.BlockSpec((_BLK_B, 128), lambda i: (i, 0)),
        out_shape=jax.ShapeDtypeStruct((_G_PAD, 128), jnp.float32),
    )(sums, cnts, ws0, bs0.reshape(1, _D), ws1, bs1.reshape(1, _D),
      wo1, bo1.reshape(1, 256), wo2p, bo2p)


def kernel(h_subgraph, subgraph_idx_batch, W_fc0, b_fc0, W_fc1, b_fc1,
           W_sum0, b_sum0, W_sum1, b_sum1, W_out1, b_out1, W_out2, b_out2):
    act = _dense(h_subgraph, W_fc0, b_fc0, W_fc1, b_fc1)

    idx_pad = jnp.concatenate(
        [subgraph_idx_batch,
         jnp.full((_N_PAD - _N,), _PAD_SEG, jnp.int32)])
    idx2d = idx_pad.reshape(_N_PAD // 128, 128)
    sums, cnts = _segment_sums(act, idx2d)

    wo2p = jnp.zeros((256, 128), jnp.float32).at[:, :_T].set(W_out2)
    bo2p = jnp.zeros((1, 128), jnp.float32).at[0, :_T].set(b_out2)
    out = _head(sums, cnts, W_sum0, b_sum0, W_sum1, b_sum1,
                W_out1, b_out1, wo2p, bo2p)
    return out[:_G, :_T]


# speedup vs baseline: 3.9610x; 1.0444x over previous
import jax
import jax.numpy as jnp
from jax import lax
from jax.experimental import pallas as pl
from jax.experimental.pallas import tpu as pltpu
from jax.experimental.pallas import tpu_sc as plsc

_N = 320000
_D = 128
_G = 5000
_T = 10

_G_PAD = 5120
_PAD_SEG = _G
_N_PAD = 327680
_K = 5
_CHUNK = _N_PAD // _K
_BLK_A = 2048
_CH = 128
_BLK_B = 512


def _elu(x):
    return jnp.where(x > 0.0, x, jnp.exp(jnp.minimum(x, 0.0)) - 1.0)


def _dense_body(x_ref, w0_ref, b0_ref, w1_ref, b1_ref, out_ref):
    x = x_ref[...].astype(jnp.bfloat16)
    w0 = w0_ref[...].astype(jnp.bfloat16)
    a = jnp.dot(x, w0, preferred_element_type=jnp.float32)
    a = _elu(a + b0_ref[...])
    w1 = w1_ref[...].astype(jnp.bfloat16)
    a = jnp.dot(a.astype(jnp.bfloat16), w1, preferred_element_type=jnp.float32)
    out_ref[...] = _elu(a + b1_ref[...])


def _dense_chunk(h, w0, b0, w1, b1, k):
    nblk_in = (_N + _BLK_A - 1) // _BLK_A
    grid = _CHUNK // _BLK_A
    return pl.pallas_call(
        _dense_body,
        grid=(grid,),
        in_specs=[
            pl.BlockSpec((_BLK_A, _D),
                         lambda i, k=k: (jnp.minimum(k * grid + i, nblk_in - 1), 0)),
            pl.BlockSpec((_D, _D), lambda i: (0, 0)),
            pl.BlockSpec((1, _D), lambda i: (0, 0)),
            pl.BlockSpec((_D, _D), lambda i: (0, 0)),
            pl.BlockSpec((1, _D), lambda i: (0, 0)),
        ],
        out_specs=pl.BlockSpec((_BLK_A, _D), lambda i: (i, 0)),
        out_shape=jax.ShapeDtypeStruct((_CHUNK, _D), jnp.float32),
    )(h, w0, b0.reshape(1, _D), w1, b1.reshape(1, _D))


def _seg_body(act, idx2d, zsum, ones, out_sum, out_cnt,
              shsum, shcnt, buf_a, buf_b, idx_v, ones_v,
              sem_a, sem_b):
    c = lax.axis_index("c")
    s = lax.axis_index("s")
    rows_per_tile = _CHUNK // 32
    base = (c * 16 + s) * rows_per_tile
    gsl = pl.ds(s * (_G_PAD // 16), _G_PAD // 16)

    pltpu.sync_copy(zsum, shsum.at[gsl])
    pltpu.sync_copy(zsum, shcnt.at[gsl])
    pltpu.sync_copy(ones, ones_v)
    plsc.subcore_barrier()

    def scatter(buf, q):
        pltpu.sync_copy(buf, shsum.at[idx_v.at[q]], add=True)
        pltpu.sync_copy(ones_v, shcnt.at[idx_v.at[q, pl.ds(0, 64)]], add=True)
        pltpu.sync_copy(ones_v, shcnt.at[idx_v.at[q, pl.ds(64, 64)]], add=True)

    npairs = rows_per_tile // (2 * _CH)

    pltpu.async_copy(act.at[pl.ds(pl.multiple_of(base, 8), _CH)], buf_a, sem_a)

    def pair_body(p, carry):
        off = pl.multiple_of(base + p * (2 * _CH), 8)

        @pl.when(lax.rem(p, 4) == 0)
        def _():
            irow = pl.multiple_of(base // 128 + (p // 4) * 8, 8)
            pltpu.sync_copy(idx2d.at[pl.ds(irow, 8)], idx_v)

        q = lax.rem(p, 4) * 2
        pltpu.make_async_copy(act.at[pl.ds(off, _CH)], buf_a, sem_a).wait()
        pltpu.async_copy(act.at[pl.ds(off + _CH, _CH)], buf_b, sem_b)
        scatter(buf_a, q)
        pltpu.make_async_copy(act.at[pl.ds(off, _CH)], buf_b, sem_b).wait()

        @pl.when(p < npairs - 1)
        def _():
            nxt = pl.multiple_of(off + 2 * _CH, 8)
            pltpu.async_copy(act.at[pl.ds(nxt, _CH)], buf_a, sem_a)

        scatter(buf_b, q + 1)
        return carry

    lax.fori_loop(0, npairs, pair_body, 0)
    plsc.subcore_barrier()

    pltpu.sync_copy(shsum.at[gsl], out_sum.at[c, gsl])
    pltpu.sync_copy(shcnt.at[gsl], out_cnt.at[c, gsl])


def _segment_sums(act, idx2d, zsum, ones):
    mesh = plsc.VectorSubcoreMesh(core_axis_name="c", subcore_axis_name="s")
    return pl.kernel(
        _seg_body,
        out_type=[
            jax.ShapeDtypeStruct((2, _G_PAD, _D), jnp.float32),
            jax.ShapeDtypeStruct((2, _G_PAD, _D), jnp.float32),
        ],
        mesh=mesh,
        scratch_types=[
            pltpu.VMEM_SHARED((_G_PAD, _D), jnp.float32),
            pltpu.VMEM_SHARED((_G_PAD, _D), jnp.float32),
            pltpu.VMEM((_CH, _D), jnp.float32),
            pltpu.VMEM((_CH, _D), jnp.float32),
            pltpu.VMEM((8, 128), jnp.int32),
            pltpu.VMEM((64, _D), jnp.float32),
            pltpu.SemaphoreType.DMA,
            pltpu.SemaphoreType.DMA,
        ],
    )(act, idx2d, zsum, ones)


def _head_body(*refs):
    sum_refs = refs[:_K]
    cnt_refs = refs[_K:2 * _K]
    (ws0_ref, bs0_ref, ws1_ref, bs1_ref,
     wo1_ref, bo1_ref, wo2_ref, bo2_ref, out_ref) = refs[2 * _K:]
    ssum = sum_refs[0][0] + sum_refs[0][1]
    cnt = cnt_refs[0][0] + cnt_refs[0][1]
    for k in range(1, _K):
        ssum = ssum + sum_refs[k][0] + sum_refs[k][1]
        cnt = cnt + cnt_refs[k][0] + cnt_refs[k][1]
    mean = ssum / jnp.maximum(cnt, 1.0)
    g = jnp.dot(mean, ws0_ref[...], preferred_element_type=jnp.float32)
    g = _elu(g + bs0_ref[...])
    g = jnp.dot(g, ws1_ref[...], preferred_element_type=jnp.float32)
    g = _elu(g + bs1_ref[...])
    r = jnp.dot(g, wo1_ref[...], preferred_element_type=jnp.float32)
    r = jnp.maximum(r + bo1_ref[...], 0.0)
    o = jnp.dot(r, wo2_ref[...], preferred_element_type=jnp.float32)
    out_ref[...] = o + bo2_ref[...]


def _head(sums, cnts, ws0, bs0, ws1, bs1, wo1, bo1, wo2p, bo2p):
    grid = _G_PAD // _BLK_B
    part_spec = pl.BlockSpec((2, _BLK_B, _D), lambda i: (0, i, 0))
    return pl.pallas_call(
        _head_body,
        grid=(grid,),
        in_specs=[part_spec] * (2 * _K) + [
            pl.BlockSpec((_D, _D), lambda i: (0, 0)),
            pl.BlockSpec((1, _D), lambda i: (0, 0)),
            pl.BlockSpec((_D, _D), lambda i: (0, 0)),
            pl.BlockSpec((1, _D), lambda i: (0, 0)),
            pl.BlockSpec((_D, 256), lambda i: (0, 0)),
            pl.BlockSpec((1, 256), lambda i: (0, 0)),
            pl.BlockSpec((256, 128), lambda i: (0, 0)),
            pl.BlockSpec((1, 128), lambda i: (0, 0)),
        ],
        out_specs=pl.BlockSpec((_BLK_B, 128), lambda i: (i, 0)),
        out_shape=jax.ShapeDtypeStruct((_G_PAD, 128), jnp.float32),
    )(*sums, *cnts, ws0, bs0.reshape(1, _D), ws1, bs1.reshape(1, _D),
      wo1, bo1.reshape(1, 256), wo2p, bo2p)


def kernel(h_subgraph, subgraph_idx_batch, W_fc0, b_fc0, W_fc1, b_fc1,
           W_sum0, b_sum0, W_sum1, b_sum1, W_out1, b_out1, W_out2, b_out2):
    idx_pad = jnp.concatenate(
        [subgraph_idx_batch,
         jnp.full((_N_PAD - _N,), _PAD_SEG, jnp.int32)])
    idx2d = idx_pad.reshape(_N_PAD // 128, 128)
    zsum = jnp.zeros((_G_PAD // 16, _D), jnp.float32)
    ones = jnp.ones((64, _D), jnp.float32)

    sums, cnts = [], []
    for k in range(_K):
        act_k = _dense_chunk(h_subgraph, W_fc0, b_fc0, W_fc1, b_fc1, k)
        idx_k = lax.slice_in_dim(idx2d, k * (_CHUNK // 128),
                                 (k + 1) * (_CHUNK // 128))
        s_k, c_k = _segment_sums(act_k, idx_k, zsum, ones)
        sums.append(s_k)
        cnts.append(c_k)

    wo2p = jnp.zeros((256, 128), jnp.float32).at[:, :_T].set(W_out2)
    bo2p = jnp.zeros((1, 128), jnp.float32).at[0, :_T].set(b_out2)
    out = _head(sums, cnts, W_sum0, b_sum0, W_sum1, b_sum1,
                W_out1, b_out1, wo2p, bo2p)
    return out[:_G, :_T]
